# Initial kernel scaffold; baseline (speedup 1.0000x reference)
import jax
import jax.numpy as jnp
from jax.experimental import pallas as pl


def kernel(x, emb_table, fc_w, fc_b):
    raise NotImplementedError("write your pallas kernel here")


# speedup vs baseline: 120.9346x; 120.9346x over previous
import functools

import jax
import jax.numpy as jnp
from jax import lax
from jax.experimental import pallas as pl
from jax.experimental.pallas import tpu as pltpu
from jax.experimental.pallas import tpu_sc as plsc

_B = 16384
_LSEQ = 200
_V = 1000
_VPAD = 1024
_D = 16
_NOUT = 3

_NC = 2
_NS = 16
_NW = _NC * _NS
_RPW = _B // _NW
_GROUPS = _RPW // 16


def _tab_kernel(emb_ref, w_ref, b_ref, out_ref):
    t = lax.dot_general(
        w_ref[...], emb_ref[...], (((1,), (1,)), ((), ())),
        preferred_element_type=jnp.float32,
    )
    out_ref[...] = (t + b_ref[:, :1]) * (1.0 / _LSEQ)


_sc_mesh = plsc.VectorSubcoreMesh(core_axis_name="c", subcore_axis_name="s")


@functools.partial(
    pl.kernel,
    mesh=_sc_mesh,
    out_type=jax.ShapeDtypeStruct((_B * _NOUT,), jnp.float32),
    scratch_types=[
        pltpu.VMEM((_NOUT, _VPAD), jnp.float32),
        pltpu.VMEM((_RPW * _LSEQ,), jnp.int32),
        pltpu.VMEM((_RPW * _NOUT,), jnp.float32),
    ],
    compiler_params=pltpu.CompilerParams(needs_layout_passes=False),
)
def _sc_pool(x_hbm, tab_hbm, out_hbm, tab_v, x_v, out_v):
    wid = lax.axis_index("s") * _NC + lax.axis_index("c")
    xbase = wid * (_RPW * _LSEQ)
    pltpu.sync_copy(tab_hbm, tab_v)
    pltpu.sync_copy(x_hbm.at[pl.ds(xbase, _RPW * _LSEQ)], x_v)

    lanes = lax.iota(jnp.int32, _NS)
    row_off = lanes * _LSEQ
    store_idx = lanes * _NOUT
    j0 = jnp.zeros((16,), jnp.int32)
    j1 = jnp.full((16,), 1, dtype=jnp.int32)
    j2 = jnp.full((16,), 2, dtype=jnp.int32)
    zero = jnp.zeros((16,), jnp.float32)

    def group_body(g, carry):
        gbase = row_off + g * (16 * _LSEQ)

        def l_body(l, accs):
            a0, a1, a2 = accs
            xi = plsc.load_gather(x_v, [gbase + l])
            a0 = a0 + plsc.load_gather(tab_v, [j0, xi])
            a1 = a1 + plsc.load_gather(tab_v, [j1, xi])
            a2 = a2 + plsc.load_gather(tab_v, [j2, xi])
            return (a0, a1, a2)

        a0, a1, a2 = lax.fori_loop(0, _LSEQ, l_body, (zero, zero, zero))
        obase = store_idx + g * (16 * _NOUT)
        plsc.store_scatter(out_v, [obase], a0)
        plsc.store_scatter(out_v, [obase + 1], a1)
        plsc.store_scatter(out_v, [obase + 2], a2)
        return carry

    lax.fori_loop(0, _GROUPS, group_body, 0)
    pltpu.sync_copy(out_v, out_hbm.at[pl.ds(wid * (_RPW * _NOUT), _RPW * _NOUT)])


def kernel(x, emb_table, fc_w, fc_b):
    xf = x.reshape(-1).astype(jnp.int32)
    emb_p = jnp.zeros((_VPAD, _D), jnp.float32).at[:_V].set(emb_table)
    w_p = jnp.zeros((8, _D), jnp.float32).at[:_NOUT].set(fc_w)
    b_p = jnp.zeros((8, 128), jnp.float32).at[:_NOUT, :].set(fc_b[:, None])
    tab = pl.pallas_call(
        _tab_kernel,
        out_shape=jax.ShapeDtypeStruct((8, _VPAD), jnp.float32),
    )(emb_p, w_p, b_p)
    out_flat = _sc_pool(xf, tab[:_NOUT])
    return out_flat.reshape(_B, _NOUT)
